# Initial kernel scaffold; baseline (speedup 1.0000x reference)
import jax
import jax.numpy as jnp
from jax.experimental import pallas as pl


def kernel(seqs, att_mask, word_embedding):
    raise NotImplementedError("write your pallas kernel here")


# speedup vs baseline: 4.1177x; 4.1177x over previous
import functools

import jax
import jax.numpy as jnp
from jax import lax
from jax.experimental import pallas as pl
from jax.experimental.pallas import tpu as pltpu
from jax.experimental.pallas import tpu_sc as plsc

NC = 2
NS = 16
NW = NC * NS
G = 128


@functools.cache
def _make_gather(N: int, V: int, D: int):
    assert N % (NW * G) == 0
    n_g = N // (NW * G)
    mesh = plsc.VectorSubcoreMesh(core_axis_name="c", subcore_axis_name="s")

    @functools.partial(
        pl.kernel,
        mesh=mesh,
        out_type=jax.ShapeDtypeStruct((N, D), jnp.float32),
        compiler_params=pltpu.CompilerParams(use_tc_tiling_on_sc=False),
        scratch_types=[
            pltpu.VMEM((n_g, G), jnp.int32),
            pltpu.VMEM((2, G, D), jnp.float32),
            pltpu.SemaphoreType.DMA,
            pltpu.SemaphoreType.DMA,
        ],
    )
    def gather_kernel(table_hbm, idx_hbm, out_hbm, idx_v, rows_v, sem0, sem1):
        wid = lax.axis_index("s") * NC + lax.axis_index("c")
        row0 = wid * n_g

        pltpu.sync_copy(idx_hbm.at[pl.ds(row0, n_g)], idx_v)

        sems = (sem0, sem1)

        for b in range(2):
            pltpu.async_copy(table_hbm.at[idx_v.at[b]], rows_v.at[b], sems[b])

        def body(it, _):
            i0 = it * 2
            for b in range(2):
                i = i0 + b
                pltpu.make_async_copy(
                    table_hbm.at[idx_v.at[i]], rows_v.at[b], sems[b]
                ).wait()
                pltpu.sync_copy(
                    rows_v.at[b], out_hbm.at[pl.ds((row0 + i) * G, G)]
                )

                @pl.when(i + 2 < n_g)
                def _():
                    pltpu.async_copy(
                        table_hbm.at[idx_v.at[i + 2]], rows_v.at[b], sems[b]
                    )
            return 0

        lax.fori_loop(0, n_g // 2, body, 0)

    return gather_kernel


def kernel(seqs, att_mask, word_embedding):
    B, L = seqs.shape
    V, D = word_embedding.shape
    N = B * L
    idx2 = seqs.reshape(N // G, G).astype(jnp.int32)
    out = _make_gather(N, V, D)(word_embedding, idx2)
    return out.reshape(B, L, D)


# speedup vs baseline: 4.2573x; 1.0339x over previous
import functools

import jax
import jax.numpy as jnp
from jax import lax
from jax.experimental import pallas as pl
from jax.experimental.pallas import tpu as pltpu
from jax.experimental.pallas import tpu_sc as plsc

NC = 2
NS = 16
NW = NC * NS
G = 128
K = 4


@functools.cache
def _make_gather(N: int, V: int, D: int):
    assert N % (NW * G * K) == 0
    n_g = N // (NW * G)
    n_s = n_g // K
    assert n_s % 2 == 0
    mesh = plsc.VectorSubcoreMesh(core_axis_name="c", subcore_axis_name="s")

    @functools.partial(
        pl.kernel,
        mesh=mesh,
        out_type=jax.ShapeDtypeStruct((N, D), jnp.float32),
        compiler_params=pltpu.CompilerParams(use_tc_tiling_on_sc=False),
        scratch_types=[
            pltpu.VMEM((n_g, G), jnp.int32),
            pltpu.VMEM((2, K * G, D), jnp.float32),
            pltpu.SemaphoreType.DMA,
            pltpu.SemaphoreType.DMA,
            pltpu.SemaphoreType.DMA,
            pltpu.SemaphoreType.DMA,
        ],
    )
    def gather_kernel(table_hbm, idx_hbm, out_hbm, idx_v, slab_v,
                      gsem0, gsem1, osem0, osem1):
        wid = lax.axis_index("s") * NC + lax.axis_index("c")
        row0 = wid * n_g

        pltpu.sync_copy(idx_hbm.at[pl.ds(row0, n_g)], idx_v)

        gsems = (gsem0, gsem1)
        osems = (osem0, osem1)

        def fill(s, p):
            for k in range(K):
                pltpu.async_copy(
                    table_hbm.at[idx_v.at[s * K + k]],
                    slab_v.at[p, pl.ds(k * G, G)],
                    gsems[p],
                )

        def drain(s, p):
            for k in range(K):
                pltpu.make_async_copy(
                    table_hbm.at[idx_v.at[s * K + k]],
                    slab_v.at[p, pl.ds(k * G, G)],
                    gsems[p],
                ).wait()

        def out_slice(s):
            return out_hbm.at[pl.ds((row0 + s * K) * G, K * G)]

        fill(0, 0)

        def body(t, _):
            for p in range(2):
                s = t * 2 + p
                q = 1 - p

                @pl.when(s + 1 < n_s)
                def _():
                    @pl.when(s >= 1)
                    def _():
                        pltpu.make_async_copy(
                            slab_v.at[q], out_slice(s - 1), osems[q]
                        ).wait()
                    fill(s + 1, q)

                drain(s, p)
                pltpu.async_copy(slab_v.at[p], out_slice(s), osems[p])
            return 0

        lax.fori_loop(0, n_s // 2, body, 0)

        pltpu.make_async_copy(slab_v.at[0], out_slice(n_s - 2), osems[0]).wait()
        pltpu.make_async_copy(slab_v.at[1], out_slice(n_s - 1), osems[1]).wait()

    return gather_kernel


def kernel(seqs, att_mask, word_embedding):
    B, L = seqs.shape
    V, D = word_embedding.shape
    N = B * L
    idx2 = seqs.reshape(N // G, G).astype(jnp.int32)
    out = _make_gather(N, V, D)(word_embedding, idx2)
    return out.reshape(B, L, D)
